# baseline (speedup 1.0000x reference)
import functools

import jax
import jax.numpy as jnp
from jax import lax
from jax.experimental import pallas as pl
from jax.experimental.pallas import tpu as pltpu
from jax.experimental.pallas import tpu_sc as plsc

D_MODEL = 1024
NUM_CORES = 2
NUM_SUBCORES = 16
NUM_WORKERS = NUM_CORES * NUM_SUBCORES

CHUNK = 32


def _make_sc_lookup(batch: int):
    assert batch % (8 * NUM_WORKERS) == 0
    b_per_w = batch // NUM_WORKERS
    assert b_per_w % CHUNK == 0
    n_chunks = b_per_w // CHUNK

    mesh = plsc.VectorSubcoreMesh(
        core_axis_name="c", subcore_axis_name="s", num_cores=NUM_CORES
    )

    @functools.partial(
        pl.kernel,
        mesh=mesh,
        out_type=jax.ShapeDtypeStruct((batch, D_MODEL), jnp.float32),
        scratch_types=[
            pltpu.VMEM((b_per_w,), jnp.int32),
            pltpu.VMEM((CHUNK, D_MODEL), jnp.float32),
        ],
    )
    def lookup(ids_hbm, table_hbm, out_hbm, idx_v, rows_v):
        wid = lax.axis_index("s") * NUM_CORES + lax.axis_index("c")
        base = wid * b_per_w
        pltpu.sync_copy(ids_hbm.at[pl.ds(base, b_per_w)], idx_v)

        def step(t, _):
            pltpu.sync_copy(
                table_hbm.at[idx_v.at[pl.ds(t * CHUNK, CHUNK)]], rows_v
            )
            pltpu.sync_copy(
                rows_v, out_hbm.at[pl.ds(base + t * CHUNK, CHUNK)]
            )
            return ()

        lax.fori_loop(0, n_chunks, step, (), unroll=False)

    return lookup


def kernel(segment_ids, table):
    b, s = segment_ids.shape
    ids_flat = segment_ids.reshape(b * s).astype(jnp.int32)
    out_flat = _make_sc_lookup(b * s)(ids_flat, table)
    return out_flat.reshape(b, s, D_MODEL)


# speedup vs baseline: 1.0006x; 1.0006x over previous
import functools

import jax
import jax.numpy as jnp
from jax import lax
from jax.experimental import pallas as pl
from jax.experimental.pallas import tpu as pltpu
from jax.experimental.pallas import tpu_sc as plsc

D_MODEL = 1024
NUM_CORES = 2
NUM_SUBCORES = 16
NUM_WORKERS = NUM_CORES * NUM_SUBCORES

CHUNK = 32
NBUF = 2


def _make_sc_lookup(batch: int):
    assert batch % (8 * NUM_WORKERS) == 0
    b_per_w = batch // NUM_WORKERS
    assert b_per_w % (CHUNK * NBUF) == 0
    n_chunks = b_per_w // CHUNK
    n_outer = n_chunks // NBUF

    mesh = plsc.VectorSubcoreMesh(
        core_axis_name="c", subcore_axis_name="s", num_cores=NUM_CORES
    )

    @functools.partial(
        pl.kernel,
        mesh=mesh,
        out_type=jax.ShapeDtypeStruct((batch, D_MODEL), jnp.float32),
        scratch_types=[
            pltpu.VMEM((b_per_w,), jnp.int32),
            pltpu.VMEM((NBUF, CHUNK, D_MODEL), jnp.float32),
            pltpu.SemaphoreType.DMA((NBUF,)),
            pltpu.SemaphoreType.DMA((NBUF,)),
        ],
    )
    def lookup(ids_hbm, table_hbm, out_hbm, idx_v, rows_v, gsems, wsems):
        wid = lax.axis_index("s") * NUM_CORES + lax.axis_index("c")
        base = wid * b_per_w
        pltpu.sync_copy(ids_hbm.at[pl.ds(base, b_per_w)], idx_v)

        def gather_copy(t, b):
            return pltpu.make_async_copy(
                table_hbm.at[idx_v.at[pl.ds(t * CHUNK, CHUNK)]],
                rows_v.at[b],
                gsems.at[b],
            )

        def write_copy(t, b):
            return pltpu.make_async_copy(
                rows_v.at[b],
                out_hbm.at[pl.ds(base + t * CHUNK, CHUNK)],
                wsems.at[b],
            )

        gather_copy(0, 0).start()

        def outer(o, _):
            for b in range(NBUF):
                t = o * NBUF + b
                gather_copy(t, b).wait()
                write_copy(t, b).start()

                @pl.when(t >= 1)
                def _():
                    write_copy(t - 1, 1 - b).wait()

                @pl.when(t + 1 < n_chunks)
                def _():
                    gather_copy(t + 1, 1 - b).start()
            return ()

        lax.fori_loop(0, n_outer, outer, (), unroll=False)
        write_copy(n_chunks - 1, (n_chunks - 1) % NBUF).wait()

    return lookup


def kernel(segment_ids, table):
    b, s = segment_ids.shape
    ids_flat = segment_ids.reshape(b * s).astype(jnp.int32)
    out_flat = _make_sc_lookup(b * s)(ids_flat, table)
    return out_flat.reshape(b, s, D_MODEL)


# speedup vs baseline: 2.0154x; 2.0141x over previous
import functools

import jax
import jax.numpy as jnp
from jax import lax
from jax.experimental import pallas as pl
from jax.experimental.pallas import tpu as pltpu
from jax.experimental.pallas import tpu_sc as plsc

D_MODEL = 1024
NUM_ROWS = 4
NUM_CORES = 2
NUM_SUBCORES = 16
NUM_WORKERS = NUM_CORES * NUM_SUBCORES
LANES = 16

CHUNK = 32
NBUF = 2


def _make_sc_lookup(batch: int):
    assert batch % (8 * NUM_WORKERS) == 0
    b_per_w = batch // NUM_WORKERS
    assert b_per_w % (CHUNK * NBUF) == 0
    n_chunks = b_per_w // CHUNK
    n_outer = n_chunks // NBUF

    mesh = plsc.VectorSubcoreMesh(
        core_axis_name="c", subcore_axis_name="s", num_cores=NUM_CORES
    )

    @functools.partial(
        pl.kernel,
        mesh=mesh,
        compiler_params=pltpu.CompilerParams(needs_layout_passes=False),
        out_type=jax.ShapeDtypeStruct((batch, D_MODEL), jnp.float32),
        scratch_types=[
            pltpu.VMEM((b_per_w,), jnp.int32),
            pltpu.VMEM((NUM_ROWS * D_MODEL,), jnp.float32),
            pltpu.VMEM((NBUF, CHUNK, D_MODEL), jnp.float32),
            pltpu.SemaphoreType.DMA((NBUF,)),
        ],
    )
    def lookup(ids_hbm, table_hbm, out_hbm, idx_v, table_v, rows_v, wsems):
        wid = lax.axis_index("s") * NUM_CORES + lax.axis_index("c")
        base = wid * b_per_w
        pltpu.sync_copy(table_hbm, table_v)
        pltpu.sync_copy(ids_hbm.at[pl.ds(base, b_per_w)], idx_v)
        lane = lax.iota(jnp.int32, LANES)

        def write_copy(t, b):
            return pltpu.make_async_copy(
                rows_v.at[b],
                out_hbm.at[pl.ds(base + t * CHUNK, CHUNK)],
                wsems.at[b],
            )

        def expand(t, b):
            def row_body(r, _):
                rid = plsc.load_gather(
                    idx_v, [jnp.full((LANES,), t * CHUNK + r, jnp.int32)]
                )
                src = rid * D_MODEL + lane
                for c in range(D_MODEL // LANES):
                    vals = plsc.load_gather(table_v, [src + c * LANES])
                    rows_v[b, r, pl.ds(c * LANES, LANES)] = vals
                return ()

            lax.fori_loop(0, CHUNK, row_body, (), unroll=False)

        def outer(o, _):
            for b in range(NBUF):
                t = o * NBUF + b

                @pl.when(o > 0)
                def _():
                    write_copy(t - NBUF, b).wait()

                expand(t, b)
                write_copy(t, b).start()
            return ()

        lax.fori_loop(0, n_outer, outer, (), unroll=False)
        for b in range(NBUF):
            write_copy((n_outer - 1) * NBUF + b, b).wait()

    return lookup


def kernel(segment_ids, table):
    b, s = segment_ids.shape
    ids_flat = segment_ids.reshape(b * s).astype(jnp.int32)
    out_flat = _make_sc_lookup(b * s)(ids_flat, table.reshape(-1))
    return out_flat.reshape(b, s, D_MODEL)


# speedup vs baseline: 8.2234x; 4.0803x over previous
import functools

import jax
import jax.numpy as jnp
from jax import lax
from jax.experimental import pallas as pl
from jax.experimental.pallas import tpu as pltpu
from jax.experimental.pallas import tpu_sc as plsc

D_MODEL = 1024
NUM_ROWS = 4
NUM_CORES = 2
NUM_SUBCORES = 16
NUM_WORKERS = NUM_CORES * NUM_SUBCORES

GROUP = 16
NSEM = 2


def _make_sc_lookup(batch: int):
    assert batch % (8 * NUM_WORKERS) == 0
    b_per_w = batch // NUM_WORKERS
    assert b_per_w % (GROUP * NSEM) == 0
    n_groups = b_per_w // GROUP

    mesh = plsc.VectorSubcoreMesh(
        core_axis_name="c", subcore_axis_name="s", num_cores=NUM_CORES
    )

    @functools.partial(
        pl.kernel,
        mesh=mesh,
        compiler_params=pltpu.CompilerParams(needs_layout_passes=False),
        out_type=jax.ShapeDtypeStruct((batch, D_MODEL), jnp.float32),
        scratch_types=[
            pltpu.VMEM((b_per_w,), jnp.int32),
            pltpu.VMEM((NUM_ROWS, D_MODEL), jnp.float32),
            pltpu.SemaphoreType.DMA((NSEM,)),
        ],
    )
    def lookup(ids_hbm, table_hbm, out_hbm, idx_v, table_v, sems):
        wid = lax.axis_index("s") * NUM_CORES + lax.axis_index("c")
        base = wid * b_per_w
        pltpu.sync_copy(table_hbm, table_v)
        pltpu.sync_copy(ids_hbm.at[pl.ds(base, b_per_w)], idx_v)

        def fire(g, s):
            idvec = idx_v[pl.ds(g * GROUP, GROUP)]
            for j in range(GROUP):
                row = g * GROUP + j
                rid = idvec[j]
                pltpu.make_async_copy(
                    table_v.at[rid],
                    out_hbm.at[base + row],
                    sems.at[s],
                ).start()

        def drain(s):
            for j in range(GROUP):
                pltpu.make_async_copy(
                    table_v.at[0],
                    out_hbm.at[0],
                    sems.at[s],
                ).wait()

        fire(0, 0)

        def outer(o, _):
            for s in range(NSEM):
                g = o * NSEM + s

                @pl.when(g + 1 < n_groups)
                def _():
                    fire(g + 1, 1 - s)

                drain(s)
            return ()

        lax.fori_loop(0, n_groups // NSEM, outer, (), unroll=False)

    return lookup


def kernel(segment_ids, table):
    b, s = segment_ids.shape
    ids_flat = segment_ids.reshape(b * s).astype(jnp.int32)
    out_flat = _make_sc_lookup(b * s)(ids_flat, table)
    return out_flat.reshape(b, s, D_MODEL)
